# baseline (speedup 1.0000x reference)
import functools

import jax
import jax.numpy as jnp
from jax import lax
from jax.experimental import pallas as pl
from jax.experimental.pallas import tpu as pltpu
from jax.experimental.pallas import tpu_sc as plsc

_E = 320000
_T = 8
_D = 128
_EPS = 1e-5

_NC = 2
_NS = 16
_NW = _NC * _NS
_BPW = _E // _NW
_C = 40
_NCHUNK = _BPW // _C
_NBUF = 10
_OUTER = _NCHUNK // _NBUF
_L = 16

_mesh = plsc.VectorSubcoreMesh(core_axis_name="c", subcore_axis_name="s")

_SPLAT_DNUMS = lax.GatherDimensionNumbers(
    offset_dims=(), collapsed_slice_dims=(0,), start_index_map=(0,))


def _splat(v, l):
    return lax.gather(
        v, jnp.full((_L, 1), l, jnp.int32), _SPLAT_DNUMS, (1,),
        mode=lax.GatherScatterMode.PROMISE_IN_BOUNDS)


def _row_total(v):
    return _splat(lax.cumsum(v, axis=0), 15)


def _rsqrt16(x):
    one = jnp.full((_L,), 1, jnp.int32)
    i = plsc.bitcast(x, jnp.int32)
    i = jnp.int32(0x5F3759DF) - lax.shift_right_logical(i, one)
    y = plsc.bitcast(i, jnp.float32)
    for _ in range(4):
        y = y * (1.5 - 0.5 * x * y * y)
    return y


@functools.partial(
    pl.kernel,
    mesh=_mesh,
    out_type=jax.ShapeDtypeStruct((_E, _D), jnp.float32),
    compiler_params=pltpu.CompilerParams(needs_layout_passes=False),
    scratch_types=[
        pltpu.VMEM((_T, _D), jnp.float32),
        pltpu.VMEM((_T, _D), jnp.float32),
        pltpu.VMEM((_T, _D), jnp.float32),
        pltpu.VMEM((_T, _D), jnp.float32),
        pltpu.VMEM_SHARED((_NS * _T, _D), jnp.float32),
        pltpu.VMEM((_BPW,), jnp.int32),
        pltpu.VMEM((_NBUF * _C, _D), jnp.float32),
        pltpu.SemaphoreType.DMA((_NBUF,)),
        pltpu.SemaphoreType.DMA((_NBUF,)),
    ],
)
def _expand(ids_hbm, tab_hbm, gam_hbm, bet_hbm, out_hbm,
            tab_v, gam_v, bet_v, p_v, p_sh, idx_v, rows_v, gsem, ssem):
    cid = lax.axis_index("c")
    sid = lax.axis_index("s")
    wid = sid * _NC + cid
    base = wid * _BPW

    pltpu.sync_copy(tab_hbm, tab_v)
    pltpu.sync_copy(gam_hbm, gam_v)
    pltpu.sync_copy(bet_hbm, bet_v)
    pltpu.sync_copy(ids_hbm.at[pl.ds(base, _BPW)], idx_v)

    scale = float(_D) ** 0.5
    for t in range(_T):
        vs = [tab_v[t, pl.ds(c * _L, _L)] * scale for c in range(_D // _L)]
        tot = vs[0]
        for c in range(1, _D // _L):
            tot = tot + vs[c]
        mean = _row_total(tot) * (1.0 / _D)
        cen = [v - mean for v in vs]
        sq = cen[0] * cen[0]
        for c in range(1, _D // _L):
            sq = sq + cen[c] * cen[c]
        var = _row_total(sq) * (1.0 / _D)
        ry = _rsqrt16(var + _EPS)
        for c in range(_D // _L):
            p_v[t, pl.ds(c * _L, _L)] = (
                cen[c] * ry * gam_v[t, pl.ds(c * _L, _L)]
                + bet_v[t, pl.ds(c * _L, _L)])

    pltpu.sync_copy(p_v, p_sh.at[pl.ds(sid * _T, _T)])

    shift = sid * _T

    @plsc.parallel_loop(0, _BPW // _L)
    def _adj(k):
        s = pl.multiple_of(k * _L, _L)
        idx_v[pl.ds(s, _L)] = idx_v[pl.ds(s, _L)] + shift

    def gather_copy(j, b):
        off = pl.multiple_of(j * _C, 8)
        return pltpu.make_async_copy(
            p_sh.at[idx_v.at[pl.ds(off, _C)]],
            rows_v.at[pl.ds(b * _C, _C)],
            gsem.at[b])

    def store_copy(j, b):
        off = pl.multiple_of(base + j * _C, 8)
        return pltpu.make_async_copy(
            rows_v.at[pl.ds(b * _C, _C)],
            out_hbm.at[pl.ds(off, _C)],
            ssem.at[b])

    for b in range(_NBUF):
        gather_copy(b, b).start()
    for b in range(_NBUF):
        gather_copy(b, b).wait()
        store_copy(b, b).start()

    def outer(grp, carry):
        jn = grp * _NBUF
        for b in range(_NBUF):
            store_copy(jn - _NBUF + b, b).wait()
            gather_copy(jn + b, b).start()
        for b in range(_NBUF):
            gather_copy(jn + b, b).wait()
            store_copy(jn + b, b).start()
        return carry

    lax.fori_loop(1, _OUTER, outer, 0)

    jlast = (_OUTER - 1) * _NBUF
    for b in range(_NBUF):
        store_copy(jlast + b, b).wait()


def kernel(edge_type_ids, table, gamma, beta):
    return _expand(edge_type_ids.astype(jnp.int32),
                   table.astype(jnp.float32),
                   gamma.astype(jnp.float32),
                   beta.astype(jnp.float32))


# speedup vs baseline: 1.0208x; 1.0208x over previous
import functools

import jax
import jax.numpy as jnp
from jax import lax
from jax.experimental import pallas as pl
from jax.experimental.pallas import tpu as pltpu
from jax.experimental.pallas import tpu_sc as plsc

_E = 320000
_T = 8
_D = 128
_EPS = 1e-5

_NC = 2
_NS = 16
_NW = _NC * _NS
_BPW = _E // _NW
_C = 40
_NCHUNK = _BPW // _C
_NBUF = 10
_OUTER = _NCHUNK // _NBUF
_L = 16

_mesh = plsc.VectorSubcoreMesh(core_axis_name="c", subcore_axis_name="s")

_SPLAT_DNUMS = lax.GatherDimensionNumbers(
    offset_dims=(), collapsed_slice_dims=(0,), start_index_map=(0,))


def _splat(v, l):
    return lax.gather(
        v, jnp.full((_L, 1), l, jnp.int32), _SPLAT_DNUMS, (1,),
        mode=lax.GatherScatterMode.PROMISE_IN_BOUNDS)


def _row_total(v):
    return _splat(lax.cumsum(v, axis=0), 15)


def _rsqrt16(x):
    one = jnp.full((_L,), 1, jnp.int32)
    i = plsc.bitcast(x, jnp.int32)
    i = jnp.int32(0x5F3759DF) - lax.shift_right_logical(i, one)
    y = plsc.bitcast(i, jnp.float32)
    for _ in range(4):
        y = y * (1.5 - 0.5 * x * y * y)
    return y


@functools.partial(
    pl.kernel,
    mesh=_mesh,
    out_type=jax.ShapeDtypeStruct((_E, _D), jnp.float32),
    compiler_params=pltpu.CompilerParams(needs_layout_passes=False),
    scratch_types=[
        pltpu.VMEM((_T, _D), jnp.float32),
        pltpu.VMEM((_T, _D), jnp.float32),
        pltpu.VMEM((_T, _D), jnp.float32),
        pltpu.VMEM((_T, _D), jnp.float32),
        pltpu.VMEM_SHARED((_T, _D), jnp.float32),
        pltpu.VMEM((_BPW,), jnp.int32),
        pltpu.VMEM((_NBUF * _C, _D), jnp.float32),
        pltpu.SemaphoreType.DMA((_NBUF,)),
        pltpu.SemaphoreType.DMA((_NBUF,)),
    ],
)
def _expand(ids_hbm, tab_hbm, gam_hbm, bet_hbm, out_hbm,
            tab_v, gam_v, bet_v, p_v, p_sh, idx_v, rows_v, gsem, ssem):
    cid = lax.axis_index("c")
    sid = lax.axis_index("s")
    wid = sid * _NC + cid
    base = wid * _BPW

    pltpu.sync_copy(tab_hbm, tab_v)
    pltpu.sync_copy(gam_hbm, gam_v)
    pltpu.sync_copy(bet_hbm, bet_v)
    pltpu.sync_copy(ids_hbm.at[pl.ds(base, _BPW)], idx_v)

    scale = float(_D) ** 0.5
    for t in range(_T):
        vs = [tab_v[t, pl.ds(c * _L, _L)] * scale for c in range(_D // _L)]
        tot = vs[0]
        for c in range(1, _D // _L):
            tot = tot + vs[c]
        mean = _row_total(tot) * (1.0 / _D)
        cen = [v - mean for v in vs]
        sq = cen[0] * cen[0]
        for c in range(1, _D // _L):
            sq = sq + cen[c] * cen[c]
        var = _row_total(sq) * (1.0 / _D)
        ry = _rsqrt16(var + _EPS)
        for c in range(_D // _L):
            p_v[t, pl.ds(c * _L, _L)] = (
                cen[c] * ry * gam_v[t, pl.ds(c * _L, _L)]
                + bet_v[t, pl.ds(c * _L, _L)])

    @pl.when(sid == 0)
    def _pub():
        pltpu.sync_copy(p_v, p_sh)

    plsc.subcore_barrier()

    def gather_copy(j, b):
        off = pl.multiple_of(j * _C, 8)
        return pltpu.make_async_copy(
            p_sh.at[idx_v.at[pl.ds(off, _C)]],
            rows_v.at[pl.ds(b * _C, _C)],
            gsem.at[b])

    def store_copy(j, b):
        off = pl.multiple_of(base + j * _C, 8)
        return pltpu.make_async_copy(
            rows_v.at[pl.ds(b * _C, _C)],
            out_hbm.at[pl.ds(off, _C)],
            ssem.at[b])

    for b in range(_NBUF):
        gather_copy(b, b).start()
    for b in range(_NBUF):
        gather_copy(b, b).wait()
        store_copy(b, b).start()

    def outer(grp, carry):
        jn = grp * _NBUF
        for b in range(_NBUF):
            store_copy(jn - _NBUF + b, b).wait()
            gather_copy(jn + b, b).start()
        for b in range(_NBUF):
            gather_copy(jn + b, b).wait()
            store_copy(jn + b, b).start()
        return carry

    lax.fori_loop(1, _OUTER, outer, 0)

    jlast = (_OUTER - 1) * _NBUF
    for b in range(_NBUF):
        store_copy(jlast + b, b).wait()


def kernel(edge_type_ids, table, gamma, beta):
    return _expand(edge_type_ids.astype(jnp.int32),
                   table.astype(jnp.float32),
                   gamma.astype(jnp.float32),
                   beta.astype(jnp.float32))


# speedup vs baseline: 1.0800x; 1.0579x over previous
import functools

import jax
import jax.numpy as jnp
from jax import lax
from jax.experimental import pallas as pl
from jax.experimental.pallas import tpu as pltpu
from jax.experimental.pallas import tpu_sc as plsc

_E = 320000
_T = 8
_D = 128
_EPS = 1e-5

_NC = 2
_NS = 16
_NW = _NC * _NS
_BPW = _E // _NW
_C = 40
_NCHUNK = _BPW // _C
_NBUF = 10
_OUTER = _NCHUNK // _NBUF
_L = 16


def _prep_body(table_ref, gamma_ref, beta_ref, out_ref):
    emb = table_ref[...] * (_D ** 0.5)
    mean = jnp.mean(emb, axis=-1, keepdims=True)
    cen = emb - mean
    var = jnp.mean(cen * cen, axis=-1, keepdims=True)
    out_ref[...] = cen * lax.rsqrt(var + _EPS) * gamma_ref[...] + beta_ref[...]


def _prep(table, gamma, beta):
    return pl.pallas_call(
        _prep_body,
        out_shape=jax.ShapeDtypeStruct((_T, _D), jnp.float32),
    )(table, gamma, beta)


_mesh = plsc.VectorSubcoreMesh(core_axis_name="c", subcore_axis_name="s")


@functools.partial(
    pl.kernel,
    mesh=_mesh,
    out_type=jax.ShapeDtypeStruct((_E, _D), jnp.float32),
    compiler_params=pltpu.CompilerParams(needs_layout_passes=False),
    scratch_types=[
        pltpu.VMEM((_T, _D), jnp.float32),
        pltpu.VMEM_SHARED((_T, _D), jnp.float32),
        pltpu.VMEM((_BPW,), jnp.int32),
        pltpu.VMEM((_NBUF * _C, _D), jnp.float32),
        pltpu.SemaphoreType.DMA,
        pltpu.SemaphoreType.DMA((_NBUF,)),
        pltpu.SemaphoreType.DMA((_NBUF,)),
    ],
)
def _expand(ids_hbm, p_hbm, out_hbm, p_v, p_sh, idx_v, rows_v,
            isem, gsem, ssem):
    cid = lax.axis_index("c")
    sid = lax.axis_index("s")
    wid = sid * _NC + cid
    base = wid * _BPW

    ids_cp = pltpu.make_async_copy(
        ids_hbm.at[pl.ds(base, _BPW)], idx_v, isem)
    ids_cp.start()

    @pl.when(sid == 0)
    def _pub():
        pltpu.sync_copy(p_hbm, p_v)
        pltpu.sync_copy(p_v, p_sh)

    plsc.subcore_barrier()
    ids_cp.wait()

    def gather_copy(j, b):
        off = pl.multiple_of(j * _C, 8)
        return pltpu.make_async_copy(
            p_sh.at[idx_v.at[pl.ds(off, _C)]],
            rows_v.at[pl.ds(b * _C, _C)],
            gsem.at[b])

    def store_copy(j, b):
        off = pl.multiple_of(base + j * _C, 8)
        return pltpu.make_async_copy(
            rows_v.at[pl.ds(b * _C, _C)],
            out_hbm.at[pl.ds(off, _C)],
            ssem.at[b])

    for b in range(_NBUF):
        gather_copy(b, b).start()
    for b in range(_NBUF):
        gather_copy(b, b).wait()
        store_copy(b, b).start()

    def outer(grp, carry):
        jn = grp * _NBUF
        for b in range(_NBUF):
            store_copy(jn - _NBUF + b, b).wait()
            gather_copy(jn + b, b).start()
        for b in range(_NBUF):
            gather_copy(jn + b, b).wait()
            store_copy(jn + b, b).start()
        return carry

    lax.fori_loop(1, _OUTER, outer, 0)

    jlast = (_OUTER - 1) * _NBUF
    for b in range(_NBUF):
        store_copy(jlast + b, b).wait()


def kernel(edge_type_ids, table, gamma, beta):
    p = _prep(table.astype(jnp.float32), gamma.astype(jnp.float32),
              beta.astype(jnp.float32))
    return _expand(edge_type_ids.astype(jnp.int32), p)
